# baseline (speedup 1.0000x reference)
import functools

import jax
import jax.numpy as jnp
from jax import lax
from jax.experimental import pallas as pl
from jax.experimental.pallas import tpu as pltpu
from jax.experimental.pallas import tpu_sc as plsc

_NUM_CORES = 2
_NUM_SUBCORES = 16
_NUM_WORKERS = _NUM_CORES * _NUM_SUBCORES

_NBUF = 4


@functools.partial(jax.jit, static_argnames=("nb",))
def _embedding_lookup(table, ids, nb):
    batch, hist = ids.shape
    depth = table.shape[1]
    per_worker = batch // _NUM_WORKERS
    n_chunks = per_worker // nb
    n_groups = n_chunks // _NBUF

    mesh = plsc.VectorSubcoreMesh(
        core_axis_name="c",
        subcore_axis_name="s",
        num_cores=_NUM_CORES,
        num_subcores=_NUM_SUBCORES,
    )

    @functools.partial(
        pl.kernel,
        mesh=mesh,
        out_type=jax.ShapeDtypeStruct((batch, hist, depth), table.dtype),
        scratch_types=(
            [pltpu.VMEM((nb, hist), jnp.int32) for _ in range(_NBUF)]
            + [pltpu.VMEM((nb, hist, depth), table.dtype) for _ in range(_NBUF)]
            + [pltpu.SemaphoreType.DMA((_NBUF,)),
               pltpu.SemaphoreType.DMA((_NBUF,))]
        ),
        compiler_params=pltpu.CompilerParams(use_tc_tiling_on_sc=False),
    )
    def emb_kernel(table_hbm, idx_hbm, out_hbm, *scratch):
        idx_v = scratch[:_NBUF]
        rows_v = scratch[_NBUF:2 * _NBUF]
        gsem, osem = scratch[2 * _NBUF], scratch[2 * _NBUF + 1]
        wid = lax.axis_index("s") * _NUM_CORES + lax.axis_index("c")
        base = wid * per_worker

        def start_chunk(j, b):
            boff = base + j * nb
            pltpu.sync_copy(idx_hbm.at[pl.ds(boff, nb)], idx_v[b])
            for r in range(nb):
                pltpu.async_copy(table_hbm.at[idx_v[b].at[r]],
                                 rows_v[b].at[r], gsem.at[b])

        def drain_chunk(j, b):
            for r in range(nb):
                pltpu.make_async_copy(table_hbm.at[idx_v[b].at[r]],
                                      rows_v[b].at[r], gsem.at[b]).wait()
            boff = base + j * nb
            pltpu.async_copy(rows_v[b], out_hbm.at[pl.ds(boff, nb)], osem.at[b])

        def wait_out(j, b):
            boff = base + j * nb
            pltpu.make_async_copy(rows_v[b], out_hbm.at[pl.ds(boff, nb)],
                                  osem.at[b]).wait()

        for b in range(_NBUF):
            start_chunk(b, b)

        def body(g, carry):
            for b in range(_NBUF):
                j = g * _NBUF + b
                drain_chunk(j - _NBUF, b)
                wait_out(j - _NBUF, b)
                start_chunk(j, b)
            return carry

        lax.fori_loop(1, n_groups, body, 0)

        for b in range(_NBUF):
            j = (n_groups - 1) * _NBUF + b
            drain_chunk(j, b)
        for b in range(_NBUF):
            j = (n_groups - 1) * _NBUF + b
            wait_out(j, b)

    return emb_kernel(table, ids)


def kernel(words_as_ids, embedding_table):
    ids = words_as_ids.astype(jnp.int32)
    return _embedding_lookup(embedding_table, ids, nb=4)
